# baseline (speedup 1.0000x reference)
import functools

import jax
import jax.numpy as jnp
from jax import lax
from jax.experimental import pallas as pl
from jax.experimental.pallas import tpu as pltpu
from jax.experimental.pallas import tpu_sc as plsc

VOCAB = 1000000
DIM = 64
PAD_DIM = 128
BATCH = 4096
TOKENS = 200

_INFO = plsc.get_sparse_core_info()
_NC, _NS, _NL = _INFO.num_cores, _INFO.num_subcores, _INFO.num_lanes
_NW = _NC * _NS
_TOTAL = BATCH * TOKENS
_GROUP = 128
_GROUPS_PER_W = _TOTAL // (_NW * _GROUP)
_NBUF = 4
_BLOCKS = _GROUPS_PER_W // _NBUF

_TBLOCKS = VOCAB // _GROUP
_TAIL = VOCAB - _TBLOCKS * _GROUP
_TAIL0 = _TBLOCKS * _GROUP
_TB_PER_W = (_TBLOCKS + _NW - 1) // _NW


def _transpose_body(tab_hbm, tail_hbm, pad_hbm, in_a, in_b, out_a, out_b,
                    tail_v, isems, osems):
    wid = lax.axis_index("s") * _NC + lax.axis_index("c")
    in_v = [in_a, in_b]
    out_v = [out_a, out_b]

    col_base = [
        jnp.arange(_NL, dtype=jnp.int32) + k * _NL for k in range(DIM // _NL)
    ]

    def blk_start(t):
        return (t * _NW + wid) * _GROUP

    def fire_load(t, p):
        pltpu.async_copy(tab_hbm.at[:, pl.ds(blk_start(t), _GROUP)],
                         in_v[p], isems[p])

    def wait_load(p):
        pltpu.make_async_copy(tab_hbm.at[:, pl.ds(0, _GROUP)],
                              in_v[p], isems[p]).wait()

    def fire_store(t, p):
        pltpu.async_copy(out_v[p], pad_hbm.at[pl.ds(blk_start(t), _GROUP)],
                         osems[p])

    def wait_store(p):
        pltpu.make_async_copy(out_v[p], pad_hbm.at[pl.ds(0, _GROUP)],
                              osems[p]).wait()

    def transpose_block(p):
        def col(v, carry):
            vv = jnp.full((_NL,), 0, dtype=jnp.int32) + v
            for k in range(DIM // _NL):
                vals = plsc.load_gather(in_v[p], [col_base[k], vv])
                out_v[p][v, pl.ds(k * _NL, _NL)] = vals
            return carry

        lax.fori_loop(0, _GROUP, col, 0, unroll=8)

    nblk = _TB_PER_W

    def active(t):
        return t * _NW + wid < _TBLOCKS

    @pl.when(active(0))
    def _():
        fire_load(0, 0)

    def step2(t2, carry):
        for p in range(2):
            t = t2 * 2 + p

            @pl.when(active(t))
            def _():
                wait_load(p)

                @pl.when(active(t + 1))
                def _():
                    fire_load(t + 1, 1 - p)

                @pl.when(t >= 2)
                def _():
                    wait_store(p)

                transpose_block(p)
                fire_store(t, p)

        return carry

    lax.fori_loop(0, (nblk + 1) // 2, step2, 0)
    wait_store(0)
    wait_store(1)

    @pl.when(wid == 0)
    def _():
        pltpu.sync_copy(tail_hbm, tail_v)

        def tail_col(v, carry):
            vv = jnp.full((_NL,), 0, dtype=jnp.int32) + v
            for k in range(DIM // _NL):
                vals = plsc.load_gather(tail_v, [col_base[k], vv])
                out_a[v, pl.ds(k * _NL, _NL)] = vals
            return carry

        lax.fori_loop(0, _TAIL, tail_col, 0, unroll=8)
        pltpu.sync_copy(out_a.at[pl.ds(0, _TAIL)],
                        pad_hbm.at[pl.ds(_TAIL0, _TAIL)])


@functools.partial(
    pl.kernel,
    mesh=plsc.VectorSubcoreMesh(core_axis_name="c", subcore_axis_name="s"),
    out_type=jax.ShapeDtypeStruct((VOCAB, PAD_DIM), jnp.float32),
    scratch_types=[
        pltpu.VMEM((DIM, _GROUP), jnp.float32),
        pltpu.VMEM((DIM, _GROUP), jnp.float32),
        pltpu.VMEM((_GROUP, PAD_DIM), jnp.float32),
        pltpu.VMEM((_GROUP, PAD_DIM), jnp.float32),
        pltpu.VMEM((DIM, _TAIL), jnp.float32),
        [pltpu.SemaphoreType.DMA] * 2,
        [pltpu.SemaphoreType.DMA] * 2,
    ],
    compiler_params=pltpu.CompilerParams(needs_layout_passes=False),
)
def _transpose_kernel(tab_hbm, tail_hbm, pad_hbm, in_a, in_b, out_a, out_b,
                      tail_v, isems, osems):
    _transpose_body(tab_hbm, tail_hbm, pad_hbm, in_a, in_b, out_a, out_b,
                    tail_v, isems, osems)


def _gather_body(idx_hbm, table_hbm, out_hbm, idx_v, rows_v, gsems, ssems):
    wid = lax.axis_index("s") * _NC + lax.axis_index("c")
    row0 = wid * _GROUPS_PER_W
    pltpu.sync_copy(idx_hbm.at[pl.ds(row0, _GROUPS_PER_W)], idx_v)

    def buf(b):
        return rows_v.at[pl.ds(b * _GROUP, _GROUP)]

    def buf_data(b):
        return rows_v.at[pl.ds(b * _GROUP, _GROUP), pl.ds(0, DIM)]

    def fire_gather(t, b):
        pltpu.async_copy(table_hbm.at[idx_v.at[t]], buf(b), gsems[b])

    def wait_gather(b):
        pltpu.make_async_copy(table_hbm.at[idx_v.at[0]], buf(b), gsems[b]).wait()

    def fire_store(t, b):
        pltpu.async_copy(buf_data(b),
                         out_hbm.at[pl.ds((row0 + t) * _GROUP, _GROUP),
                                    pl.ds(0, DIM)],
                         ssems[b])

    def wait_store(b):
        pltpu.make_async_copy(
            buf_data(b),
            out_hbm.at[pl.ds(row0 * _GROUP, _GROUP), pl.ds(0, DIM)],
            ssems[b]).wait()

    for b in range(_NBUF):
        fire_gather(b, b)

    def block(gi, carry):
        g = gi * _NBUF
        for b in range(_NBUF):
            t = g + b
            wait_gather(b)
            fire_store(t, b)
            wait_store(b)
            fire_gather(t + _NBUF, b)
        return carry

    lax.fori_loop(0, _BLOCKS - 1, block, 0)
    g = (_BLOCKS - 1) * _NBUF
    for b in range(_NBUF):
        wait_gather(b)
        fire_store(g + b, b)
    for b in range(_NBUF):
        wait_store(b)


@functools.partial(
    pl.kernel,
    mesh=plsc.VectorSubcoreMesh(core_axis_name="c", subcore_axis_name="s"),
    out_type=jax.ShapeDtypeStruct((_TOTAL, PAD_DIM), jnp.float32),
    scratch_types=[
        pltpu.VMEM((_GROUPS_PER_W, _GROUP), jnp.int32),
        pltpu.VMEM((_NBUF * _GROUP, PAD_DIM), jnp.float32),
        [pltpu.SemaphoreType.DMA] * _NBUF,
        [pltpu.SemaphoreType.DMA] * _NBUF,
    ],
    compiler_params=pltpu.CompilerParams(use_tc_tiling_on_sc=False),
)
def _gather_kernel(idx_hbm, table_hbm, out_hbm, idx_v, rows_v, gsems, ssems):
    _gather_body(idx_hbm, table_hbm, out_hbm, idx_v, rows_v, gsems, ssems)


def kernel(inputs, embed_weight):
    idx = inputs.astype(jnp.int32).reshape(_TOTAL // _GROUP, _GROUP)
    wt = embed_weight.T
    table = _transpose_kernel(wt, wt[:, _TAIL0:])
    out = _gather_kernel(idx, table)
    return out[:, :DIM].reshape(BATCH, TOKENS, DIM)


# speedup vs baseline: 2.0331x; 2.0331x over previous
import functools

import jax
import jax.numpy as jnp
from jax import lax
from jax.experimental import pallas as pl
from jax.experimental.pallas import tpu as pltpu
from jax.experimental.pallas import tpu_sc as plsc

VOCAB = 1000000
DIM = 64
PAD_DIM = 128
BATCH = 4096
TOKENS = 200

_INFO = plsc.get_sparse_core_info()
_NC, _NS = _INFO.num_cores, _INFO.num_subcores
_NW = _NC * _NS
_TOTAL = BATCH * TOKENS
_GROUP = 128
_GROUPS_PER_W = _TOTAL // (_NW * _GROUP)
_NBUF = 4
_BLOCKS = _GROUPS_PER_W // _NBUF


def _body(idx_hbm, table_hbm, out_hbm, idx_v, rows_v, gsems, ssems):
    wid = lax.axis_index("s") * _NC + lax.axis_index("c")
    row0 = wid * _GROUPS_PER_W
    pltpu.sync_copy(idx_hbm.at[pl.ds(row0, _GROUPS_PER_W)], idx_v)

    def buf(b):
        return rows_v.at[pl.ds(b * _GROUP, _GROUP)]

    def buf_data(b):
        return rows_v.at[pl.ds(b * _GROUP, _GROUP), pl.ds(0, DIM)]

    def fire_gather(t, b):
        pltpu.async_copy(table_hbm.at[idx_v.at[t]], buf(b), gsems[b])

    def wait_gather(b):
        pltpu.make_async_copy(table_hbm.at[idx_v.at[0]], buf(b), gsems[b]).wait()

    def fire_store(t, b):
        pltpu.async_copy(buf_data(b),
                         out_hbm.at[pl.ds((row0 + t) * _GROUP, _GROUP),
                                    pl.ds(0, DIM)],
                         ssems[b])

    def wait_store(b):
        pltpu.make_async_copy(
            buf_data(b),
            out_hbm.at[pl.ds(row0 * _GROUP, _GROUP), pl.ds(0, DIM)],
            ssems[b]).wait()

    for b in range(_NBUF):
        fire_gather(b, b)

    def block(gi, carry):
        g = gi * _NBUF
        for b in range(_NBUF):
            t = g + b
            wait_gather(b)
            fire_store(t, b)
            wait_store(b)
            fire_gather(t + _NBUF, b)
        return carry

    lax.fori_loop(0, _BLOCKS - 1, block, 0)
    g = (_BLOCKS - 1) * _NBUF
    for b in range(_NBUF):
        wait_gather(b)
        fire_store(g + b, b)
    for b in range(_NBUF):
        wait_store(b)


@functools.partial(
    pl.kernel,
    mesh=plsc.VectorSubcoreMesh(core_axis_name="c", subcore_axis_name="s"),
    out_type=jax.ShapeDtypeStruct((_TOTAL, PAD_DIM), jnp.float32),
    scratch_types=[
        pltpu.VMEM((_GROUPS_PER_W, _GROUP), jnp.int32),
        pltpu.VMEM((_NBUF * _GROUP, PAD_DIM), jnp.float32),
        [pltpu.SemaphoreType.DMA] * _NBUF,
        [pltpu.SemaphoreType.DMA] * _NBUF,
    ],
    compiler_params=pltpu.CompilerParams(use_tc_tiling_on_sc=False),
)
def _gather_kernel(idx_hbm, table_hbm, out_hbm, idx_v, rows_v, gsems, ssems):
    _body(idx_hbm, table_hbm, out_hbm, idx_v, rows_v, gsems, ssems)


def kernel(inputs, embed_weight):
    idx = inputs.astype(jnp.int32).reshape(_TOTAL // _GROUP, _GROUP)
    table = jnp.pad(embed_weight, ((0, 0), (0, PAD_DIM - DIM)))
    out = _gather_kernel(idx, table)
    return out[:, :DIM].reshape(BATCH, TOKENS, DIM)
